# baseline (speedup 1.0000x reference)
import math

import jax
import jax.numpy as jnp
from jax import lax
from jax.experimental import pallas as pl
from jax.experimental.pallas import tpu as pltpu

SIZE_ = 32000
PAD_ = 0
SMOOTH_ = 0.1
CONF_ = 1.0 - SMOOTH_
SVAL_ = SMOOTH_ / (SIZE_ - 2)
K_ = (SIZE_ - 2) * SVAL_ * math.log(SVAL_) + CONF_ * math.log(CONF_)

B_ = 2048
BR_ = 128


def _loss_body(out_ref, t_ref, acc_ref):
    j = pl.program_id(0)
    out_blk = out_ref[...]
    t_blk = t_ref[...]
    nonpad = t_blk != PAD_

    colids = lax.broadcasted_iota(jnp.int32, (BR_, SIZE_), 1)
    ones = jnp.ones((SIZE_, 1), jnp.float32)
    rowsum = jax.lax.dot_general(
        out_blk, ones, (((1,), (0,)), ((), ())),
        preferred_element_type=jnp.float32)
    pick = jax.lax.dot_general(
        jnp.where(colids == t_blk, out_blk, 0.0), ones,
        (((1,), (0,)), ((), ())), preferred_element_type=jnp.float32)
    out0 = out_blk[:, 0:1]
    per_row = K_ - SVAL_ * (rowsum - out0) - (CONF_ - SVAL_) * pick
    partial = jnp.sum(jnp.where(nonpad, per_row, 0.0))

    @pl.when(j == 0)
    def _init():
        acc_ref[0, 0] = 0.0

    acc_ref[0, 0] = acc_ref[0, 0] + partial


@jax.jit
def kernel(output, target, one_hot):
    del one_hot
    t2 = target.astype(jnp.int32).reshape(B_, 1)
    acc = pl.pallas_call(
        _loss_body,
        grid=(B_ // BR_,),
        in_specs=[
            pl.BlockSpec((BR_, SIZE_), lambda j: (j, 0)),
            pl.BlockSpec((BR_, 1), lambda j: (j, 0)),
        ],
        out_specs=pl.BlockSpec(
            (1, 1), lambda j: (0, 0), memory_space=pltpu.SMEM
        ),
        out_shape=jax.ShapeDtypeStruct((1, 1), jnp.float32),
    )(output, t2)
    return acc[0, 0]


# speedup vs baseline: 1.0356x; 1.0356x over previous
import math

import jax
import jax.numpy as jnp
from jax import lax
from jax.experimental import pallas as pl
from jax.experimental.pallas import tpu as pltpu

SIZE_ = 32000
PAD_ = 0
SMOOTH_ = 0.1
CONF_ = 1.0 - SMOOTH_
SVAL_ = SMOOTH_ / (SIZE_ - 2)
K_ = (SIZE_ - 2) * SVAL_ * math.log(SVAL_) + CONF_ * math.log(CONF_)

B_ = 2048
BR_ = 64


def _loss_body(out_ref, t_ref, acc_ref):
    j = pl.program_id(0)
    out_blk = out_ref[...]
    t_blk = t_ref[...]
    nonpad = t_blk != PAD_

    colids = lax.broadcasted_iota(jnp.int32, (BR_, SIZE_), 1)
    rowsum = jnp.sum(out_blk, axis=1, keepdims=True)
    pick = jnp.sum(jnp.where(colids == t_blk, out_blk, 0.0),
                   axis=1, keepdims=True)
    out0 = out_blk[:, 0:1]
    per_row = K_ - SVAL_ * (rowsum - out0) - (CONF_ - SVAL_) * pick
    partial = jnp.sum(jnp.where(nonpad, per_row, 0.0))

    @pl.when(j == 0)
    def _init():
        acc_ref[0, 0] = 0.0

    acc_ref[0, 0] = acc_ref[0, 0] + partial


@jax.jit
def kernel(output, target, one_hot):
    del one_hot
    t2 = target.astype(jnp.int32).reshape(B_, 1)
    acc = pl.pallas_call(
        _loss_body,
        grid=(B_ // BR_,),
        in_specs=[
            pl.BlockSpec((BR_, SIZE_), lambda j: (j, 0)),
            pl.BlockSpec((BR_, 1), lambda j: (j, 0)),
        ],
        out_specs=pl.BlockSpec(
            (1, 1), lambda j: (0, 0), memory_space=pltpu.SMEM
        ),
        out_shape=jax.ShapeDtypeStruct((1, 1), jnp.float32),
    )(output, t2)
    return acc[0, 0]
